# baseline (speedup 1.0000x reference)
import functools

import jax
import jax.numpy as jnp
from jax import lax
from jax.experimental import pallas as pl
from jax.experimental.pallas import tpu as pltpu
from jax.experimental.pallas import tpu_sc as plsc

BATCH = 4096
MAXLEN = 200
EMB = 32
LABELS = 4

NC = 2
NS = 16
NW = NC * NS
BPW = BATCH // NW
CH = 100
CPS = MAXLEN // CH
NCHUNK = BPW * CPS
NBUF = 4
NGROUP = NCHUNK // NBUF


VOCAB = 1000000
NFULL = VOCAB // 128
TAIL = VOCAB - NFULL * 128
TAIL_W = 31


def _tc_detile_pad(table):
    tableT = table.T
    C = 4096
    nblk = (VOCAB + C - 1) // C
    eye = jnp.eye(EMB, dtype=jnp.float32)

    def body(t_ref, eye_ref, o_ref):
        tt = lax.dot_general(
            t_ref[...], eye_ref[...], (((0,), (0,)), ((), ())),
            preferred_element_type=jnp.float32,
        )
        o_ref[:, 0:EMB] = tt.astype(jnp.bfloat16)

    return pl.pallas_call(
        body,
        grid=(nblk,),
        in_specs=[
            pl.BlockSpec((EMB, C), lambda i: (0, i)),
            pl.BlockSpec((EMB, EMB), lambda i: (0, 0)),
        ],
        out_specs=pl.BlockSpec((C, 128), lambda i: (i, 0)),
        out_shape=jax.ShapeDtypeStruct((VOCAB, 128), jnp.bfloat16),
    )(tableT, eye)


def _sc_detile(table):
    tableT = table.T
    mesh = plsc.VectorSubcoreMesh(core_axis_name="c", subcore_axis_name="s")

    @functools.partial(
        pl.kernel,
        mesh=mesh,
        compiler_params=pltpu.CompilerParams(use_tc_tiling_on_sc=True),
        out_type=jax.ShapeDtypeStruct((VOCAB * EMB,), jnp.float32),
        scratch_types=[
            pltpu.VMEM((2, EMB, 128), jnp.float32),
            pltpu.VMEM((2, 128 * EMB), jnp.float32),
            pltpu.SemaphoreType.DMA,
            pltpu.SemaphoreType.DMA,
            pltpu.SemaphoreType.DMA,
            pltpu.SemaphoreType.DMA,
        ],
    )
    def k(t_hbm, out_hbm, inb, outb, si0, si1, so0, so1):
        sin = (si0, si1)
        sout = (so0, so1)
        wid = lax.axis_index("s") * NC + lax.axis_index("c")
        nc = (NFULL - wid + NW - 1) // NW
        d_lo = lax.iota(jnp.int32, 16)
        d_hi = d_lo + 16

        def start_in(t, bi):
            c = wid + t * NW
            pltpu.async_copy(
                t_hbm.at[:, pl.ds(c * 128, 128)], inb.at[bi], sin[bi])

        def shuffle(bi, nrows):
            for j0 in range(0, nrows, 4):
                for u in range(4):
                    j = j0 + u
                    js = jnp.full((16,), j, jnp.int32)
                    lo = plsc.load_gather(inb.at[bi], [d_lo, js])
                    hi = plsc.load_gather(inb.at[bi], [d_hi, js])
                    outb[bi, pl.ds(j * EMB, 16)] = lo
                    outb[bi, pl.ds(j * EMB + 16, 16)] = hi

        @pl.when(nc > 0)
        def _():
            start_in(0, 0)

        @pl.when(nc > 1)
        def _():
            start_in(1, 1)

        def step(t, carry):
            bi = lax.rem(t, 2)
            for b in range(2):
                @pl.when(bi == b)
                def _(b=b):
                    c = wid + t * NW
                    pltpu.make_async_copy(
                        t_hbm.at[:, pl.ds(c * 128, 128)], inb.at[b], sin[b]
                    ).wait()

                    @pl.when(t >= 2)
                    def _(b=b):
                        co = wid + (t - 2) * NW
                        pltpu.make_async_copy(
                            outb.at[b],
                            out_hbm.at[pl.ds(co * (128 * EMB), 128 * EMB)],
                            sout[b],
                        ).wait()

                    shuffle(b, 128)
                    pltpu.async_copy(
                        outb.at[b],
                        out_hbm.at[pl.ds(c * (128 * EMB), 128 * EMB)],
                        sout[b],
                    )

                    @pl.when(t + 2 < nc)
                    def _(b=b):
                        start_in(t + 2, b)
            return carry

        lax.fori_loop(0, nc, step, 0)

        for bb in range(2):
            pltpu.make_async_copy(
                outb.at[bb],
                out_hbm.at[pl.ds(wid * (128 * EMB), 128 * EMB)],
                sout[bb],
            ).wait()

        @pl.when(wid == TAIL_W)
        def _():
            pltpu.sync_copy(t_hbm.at[:, pl.ds(NFULL * 128, TAIL)],
                            inb.at[0, :, pl.ds(0, TAIL)])
            shuffle(0, TAIL)
            pltpu.sync_copy(outb.at[0, pl.ds(0, TAIL * EMB)],
                            out_hbm.at[pl.ds(NFULL * 128 * EMB, TAIL * EMB)])

    return k(tableT)


def _sc_pool(x2d, table):
    mesh = plsc.VectorSubcoreMesh(core_axis_name="c", subcore_axis_name="s")

    @functools.partial(
        pl.kernel,
        mesh=mesh,
        compiler_params=pltpu.CompilerParams(
            use_tc_tiling_on_sc=False, needs_layout_passes=False),
        out_type=jax.ShapeDtypeStruct((BATCH, EMB), jnp.float32),
        scratch_types=[
            pltpu.VMEM((NCHUNK, CH), jnp.int32),
            pltpu.VMEM((NBUF, CH, EMB), jnp.bfloat16),
            pltpu.VMEM((BPW, EMB), jnp.float32),
            pltpu.SemaphoreType.DMA,
            pltpu.SemaphoreType.DMA,
            pltpu.SemaphoreType.DMA,
            pltpu.SemaphoreType.DMA,
        ],
    )
    def k(x_hbm, table_hbm, out_hbm, idx_v, rows_v, acc_v, s0, s1, s2, s3):
        sems = (s0, s1, s2, s3)
        wid = lax.axis_index("s") * NC + lax.axis_index("c")
        pltpu.sync_copy(x_hbm.at[pl.ds(wid * NCHUNK, NCHUNK)], idx_v)

        def start(ci, b):
            pltpu.async_copy(table_hbm.at[idx_v.at[ci]], rows_v.at[b], sems[b])

        for b in range(NBUF):
            start(b, b)

        def group(g, carry):
            for sl in range(2):
                i = g * 2 + sl
                a0 = jnp.zeros((16,), jnp.float32)
                a1 = jnp.zeros((16,), jnp.float32)
                for j in range(CPS):
                    b = sl * CPS + j
                    ci = g * NBUF + b
                    pltpu.make_async_copy(
                        table_hbm.at[idx_v.at[ci]], rows_v.at[b], sems[b]
                    ).wait()

                    def rbody(rr, c, _b=b):
                        c0, c1, c2, c3 = c
                        r = rr * 5
                        for u in range(5):
                            ev, od = plsc.unpack(
                                rows_v[_b, r + u],
                                format=plsc.PackFormat.INTERLEAVED)
                            if u % 2 == 0:
                                c0 = c0 + ev
                                c1 = c1 + od
                            else:
                                c2 = c2 + ev
                                c3 = c3 + od
                        return (c0, c1, c2, c3)

                    z = jnp.zeros((16,), jnp.float32)
                    a0, a1, a2, a3 = lax.fori_loop(
                        0, CH // 5, rbody, (a0, a1, z, z))
                    a0 = a0 + a2
                    a1 = a1 + a3

                    nci = ci + NBUF

                    @pl.when(nci < NCHUNK)
                    def _(nci=nci, b=b):
                        start(nci, b)

                acc_v[i, pl.ds(0, 16)] = a0
                acc_v[i, pl.ds(16, 16)] = a1
            return carry

        lax.fori_loop(0, NGROUP, group, 0)
        pltpu.sync_copy(acc_v, out_hbm.at[pl.ds(wid * BPW, BPW)])

    return k(x2d, table)


def _tc_classify(pooled_sum, W, b2d):
    def body(p_ref, w_ref, b_ref, o_ref):
        p = p_ref[...]
        w = w_ref[...]
        acc = lax.dot_general(
            p, w, (((1,), (1,)), ((), ())),
            preferred_element_type=jnp.float32,
        )
        o_ref[...] = acc * (1.0 / MAXLEN) + b_ref[...]

    return pl.pallas_call(
        body,
        out_shape=jax.ShapeDtypeStruct((BATCH, LABELS), jnp.float32),
    )(pooled_sum, W, b2d)


def kernel(x, table, W, b):
    x2d = (x.reshape(BATCH * CPS, CH) * 4).astype(jnp.int32)
    table_lin = _tc_detile_pad(table).reshape(4 * VOCAB, EMB)
    pooled_sum = _sc_pool(x2d, table_lin)
    W_perm = jnp.concatenate([W[:, 0::2], W[:, 1::2]], axis=1)
    return _tc_classify(pooled_sum, W_perm, b.reshape(1, LABELS))


# speedup vs baseline: 1.5076x; 1.5076x over previous
import functools

import jax
import jax.numpy as jnp
from jax import lax
from jax.experimental import pallas as pl
from jax.experimental.pallas import tpu as pltpu
from jax.experimental.pallas import tpu_sc as plsc

BATCH = 4096
MAXLEN = 200
EMB = 32
LABELS = 4

NC = 2
NS = 16
NW = NC * NS
BPW = BATCH // NW
CH = 100
CPS = MAXLEN // CH
NCHUNK = BPW * CPS
NBUF = 4
NGROUP = NCHUNK // NBUF


VOCAB = 1000000
NFULL = VOCAB // 128
TAIL = VOCAB - NFULL * 128
TAIL_W = 31


def _tc_detile_pad(table):
    tableT = table.T
    C = 4096
    nblk = (VOCAB + C - 1) // C
    eye = jnp.eye(EMB, dtype=jnp.float32)

    def body(t_ref, eye_ref, o_ref):
        tt = lax.dot_general(
            t_ref[...], eye_ref[...], (((0,), (0,)), ((), ())),
            preferred_element_type=jnp.float32,
        )
        o_ref[:, 0:EMB] = tt.astype(jnp.bfloat16)

    return pl.pallas_call(
        body,
        grid=(nblk,),
        in_specs=[
            pl.BlockSpec((EMB, C), lambda i: (0, i)),
            pl.BlockSpec((EMB, EMB), lambda i: (0, 0)),
        ],
        out_specs=pl.BlockSpec((C, 128), lambda i: (i, 0)),
        out_shape=jax.ShapeDtypeStruct((VOCAB, 128), jnp.bfloat16),
    )(tableT, eye)


def _sc_detile(table, tail_lin):
    tableT = table.T
    mesh = plsc.VectorSubcoreMesh(core_axis_name="c", subcore_axis_name="s")

    @functools.partial(
        pl.kernel,
        mesh=mesh,
        compiler_params=pltpu.CompilerParams(
            use_tc_tiling_on_sc=True, needs_layout_passes=False),
        out_type=jax.ShapeDtypeStruct((VOCAB * EMB,), jnp.float32),
        scratch_types=[
            pltpu.VMEM((2, EMB, 128), jnp.float32),
            pltpu.VMEM((2, 128 * EMB), jnp.float32),
            pltpu.SemaphoreType.DMA,
            pltpu.SemaphoreType.DMA,
            pltpu.SemaphoreType.DMA,
            pltpu.SemaphoreType.DMA,
        ],
    )
    def k(t_hbm, tail_hbm, out_hbm, inb, outb, si0, si1, so0, so1):
        sin = (si0, si1)
        sout = (so0, so1)
        wid = lax.axis_index("s") * NC + lax.axis_index("c")
        nc = (NFULL - wid + NW - 1) // NW
        d_lo = lax.iota(jnp.int32, 16)
        d_hi = d_lo + 16

        def start_in(t, bi):
            c = wid + t * NW
            pltpu.async_copy(
                t_hbm.at[:, pl.ds(c * 128, 128)], inb.at[bi], sin[bi])

        def shuffle(bi, nrows):
            for j0 in range(0, nrows, 4):
                for u in range(4):
                    j = j0 + u
                    js = jnp.full((16,), j, jnp.int32)
                    lo = plsc.load_gather(inb.at[bi], [d_lo, js])
                    hi = plsc.load_gather(inb.at[bi], [d_hi, js])
                    outb[bi, pl.ds(j * EMB, 16)] = lo
                    outb[bi, pl.ds(j * EMB + 16, 16)] = hi

        @pl.when(nc > 0)
        def _():
            start_in(0, 0)

        @pl.when(nc > 1)
        def _():
            start_in(1, 1)

        def step(t, carry):
            bi = lax.rem(t, 2)
            for b in range(2):
                @pl.when(bi == b)
                def _(b=b):
                    c = wid + t * NW
                    pltpu.make_async_copy(
                        t_hbm.at[:, pl.ds(c * 128, 128)], inb.at[b], sin[b]
                    ).wait()

                    @pl.when(t >= 2)
                    def _(b=b):
                        co = wid + (t - 2) * NW
                        pltpu.make_async_copy(
                            outb.at[b],
                            out_hbm.at[pl.ds(co * (128 * EMB), 128 * EMB)],
                            sout[b],
                        ).wait()

                    shuffle(b, 128)
                    pltpu.async_copy(
                        outb.at[b],
                        out_hbm.at[pl.ds(c * (128 * EMB), 128 * EMB)],
                        sout[b],
                    )

                    @pl.when(t + 2 < nc)
                    def _(b=b):
                        start_in(t + 2, b)
            return carry

        lax.fori_loop(0, nc, step, 0)

        for bb in range(2):
            pltpu.make_async_copy(
                outb.at[bb],
                out_hbm.at[pl.ds(wid * (128 * EMB), 128 * EMB)],
                sout[bb],
            ).wait()

        @pl.when(wid == TAIL_W)
        def _():
            pltpu.sync_copy(
                tail_hbm,
                out_hbm.at[pl.ds(NFULL * 128 * EMB, TAIL * EMB)])

    return k(tableT, tail_lin)


def _sc_pool(x2d, table):
    mesh = plsc.VectorSubcoreMesh(core_axis_name="c", subcore_axis_name="s")

    @functools.partial(
        pl.kernel,
        mesh=mesh,
        compiler_params=pltpu.CompilerParams(
            use_tc_tiling_on_sc=False, needs_layout_passes=False),
        out_type=jax.ShapeDtypeStruct((BATCH, EMB), jnp.float32),
        scratch_types=[
            pltpu.VMEM((NCHUNK, CH), jnp.int32),
            pltpu.VMEM((NBUF, CH, EMB), jnp.float32),
            pltpu.VMEM((BPW, EMB), jnp.float32),
            pltpu.SemaphoreType.DMA,
            pltpu.SemaphoreType.DMA,
            pltpu.SemaphoreType.DMA,
            pltpu.SemaphoreType.DMA,
        ],
    )
    def k(x_hbm, table_hbm, out_hbm, idx_v, rows_v, acc_v, s0, s1, s2, s3):
        sems = (s0, s1, s2, s3)
        wid = lax.axis_index("s") * NC + lax.axis_index("c")
        pltpu.sync_copy(x_hbm.at[pl.ds(wid * NCHUNK, NCHUNK)], idx_v)

        def start(ci, b):
            pltpu.async_copy(table_hbm.at[idx_v.at[ci]], rows_v.at[b], sems[b])

        for b in range(NBUF):
            start(b, b)

        def group(g, carry):
            for sl in range(2):
                i = g * 2 + sl
                a0 = jnp.zeros((16,), jnp.float32)
                a1 = jnp.zeros((16,), jnp.float32)
                for j in range(CPS):
                    b = sl * CPS + j
                    ci = g * NBUF + b
                    pltpu.make_async_copy(
                        table_hbm.at[idx_v.at[ci]], rows_v.at[b], sems[b]
                    ).wait()

                    def rbody(rr, c, _b=b):
                        c0, c1, c2, c3 = c
                        r = rr * 5
                        c0 = c0 + rows_v[_b, r, pl.ds(0, 16)]
                        c1 = c1 + rows_v[_b, r, pl.ds(16, 16)]
                        c2 = c2 + rows_v[_b, r + 1, pl.ds(0, 16)]
                        c3 = c3 + rows_v[_b, r + 1, pl.ds(16, 16)]
                        c0 = c0 + rows_v[_b, r + 2, pl.ds(0, 16)]
                        c1 = c1 + rows_v[_b, r + 2, pl.ds(16, 16)]
                        c2 = c2 + rows_v[_b, r + 3, pl.ds(0, 16)]
                        c3 = c3 + rows_v[_b, r + 3, pl.ds(16, 16)]
                        c0 = c0 + rows_v[_b, r + 4, pl.ds(0, 16)]
                        c1 = c1 + rows_v[_b, r + 4, pl.ds(16, 16)]
                        return (c0, c1, c2, c3)

                    z = jnp.zeros((16,), jnp.float32)
                    a0, a1, a2, a3 = lax.fori_loop(
                        0, CH // 5, rbody, (a0, a1, z, z))
                    a0 = a0 + a2
                    a1 = a1 + a3

                    nci = ci + NBUF

                    @pl.when(nci < NCHUNK)
                    def _(nci=nci, b=b):
                        start(nci, b)

                acc_v[i, pl.ds(0, 16)] = a0
                acc_v[i, pl.ds(16, 16)] = a1
            return carry

        lax.fori_loop(0, NGROUP, group, 0)
        pltpu.sync_copy(acc_v, out_hbm.at[pl.ds(wid * BPW, BPW)])

    return k(x2d, table)


def _tc_classify(pooled_sum, W, b2d):
    def body(p_ref, w_ref, b_ref, o_ref):
        p = p_ref[...]
        w = w_ref[...]
        acc = lax.dot_general(
            p, w, (((1,), (1,)), ((), ())),
            preferred_element_type=jnp.float32,
        )
        o_ref[...] = acc * (1.0 / MAXLEN) + b_ref[...]

    return pl.pallas_call(
        body,
        out_shape=jax.ShapeDtypeStruct((BATCH, LABELS), jnp.float32),
    )(pooled_sum, W, b2d)


def kernel(x, table, W, b):
    x2d = x.reshape(BATCH * CPS, CH).astype(jnp.int32)
    tail_lin = table[VOCAB - TAIL:, :].reshape(TAIL * EMB)
    table_lin = _sc_detile(table, tail_lin).reshape(VOCAB, EMB)
    pooled_sum = _sc_pool(x2d, table_lin)
    return _tc_classify(pooled_sum, W, b.reshape(1, LABELS))


# speedup vs baseline: 1.7097x; 1.1341x over previous
import functools

import jax
import jax.numpy as jnp
from jax import lax
from jax.experimental import pallas as pl
from jax.experimental.pallas import tpu as pltpu
from jax.experimental.pallas import tpu_sc as plsc

BATCH = 4096
MAXLEN = 200
EMB = 32
LABELS = 4

NC = 2
NS = 16
NW = NC * NS
BPW = BATCH // NW
CH = 100
CPS = MAXLEN // CH
NCHUNK = BPW * CPS
NBUF = 4
NGROUP = NCHUNK // NBUF


VOCAB = 1000000
NFULL = VOCAB // 128
TAIL = VOCAB - NFULL * 128
TAIL_W = 31


def _tc_detile_pad(table):
    tableT = table.T
    C = 4096
    nblk = (VOCAB + C - 1) // C
    eye = jnp.eye(EMB, dtype=jnp.float32)

    def body(t_ref, eye_ref, o_ref):
        tt = lax.dot_general(
            t_ref[...], eye_ref[...], (((0,), (0,)), ((), ())),
            preferred_element_type=jnp.float32,
        )
        o_ref[:, 0:EMB] = tt.astype(jnp.bfloat16)

    return pl.pallas_call(
        body,
        grid=(nblk,),
        in_specs=[
            pl.BlockSpec((EMB, C), lambda i: (0, i)),
            pl.BlockSpec((EMB, EMB), lambda i: (0, 0)),
        ],
        out_specs=pl.BlockSpec((C, 128), lambda i: (i, 0)),
        out_shape=jax.ShapeDtypeStruct((VOCAB, 128), jnp.bfloat16),
    )(tableT, eye)


def _sc_detile(table, tail_lin):
    tableT = table.T
    mesh = plsc.VectorSubcoreMesh(core_axis_name="c", subcore_axis_name="s")

    @functools.partial(
        pl.kernel,
        mesh=mesh,
        compiler_params=pltpu.CompilerParams(
            use_tc_tiling_on_sc=True, needs_layout_passes=False),
        out_type=jax.ShapeDtypeStruct((VOCAB * EMB,), jnp.float32),
        scratch_types=[
            pltpu.VMEM((EMB, 128), jnp.float32),
            pltpu.VMEM((EMB, 128), jnp.float32),
            pltpu.VMEM((128 * EMB,), jnp.float32),
            pltpu.VMEM((128 * EMB,), jnp.float32),
            pltpu.SemaphoreType.DMA,
            pltpu.SemaphoreType.DMA,
            pltpu.SemaphoreType.DMA,
            pltpu.SemaphoreType.DMA,
        ],
    )
    def k(t_hbm, tail_hbm, out_hbm, inb0, inb1, outb0, outb1,
          si0, si1, so0, so1):
        inb = (inb0, inb1)
        outb = (outb0, outb1)
        sin = (si0, si1)
        sout = (so0, so1)
        wid = lax.axis_index("s") * NC + lax.axis_index("c")
        nc = (NFULL - wid + NW - 1) // NW
        d_lo = lax.iota(jnp.int32, 16)
        d_hi = d_lo + 16

        def start_in(t, bi):
            c = wid + t * NW
            pltpu.async_copy(
                t_hbm.at[:, pl.ds(c * 128, 128)], inb[bi], sin[bi])

        iota32 = lax.iota(jnp.int32, 16) * EMB

        def shuffle(bi, nrows):
            for d in range(EMB):
                for u in range(nrows // 16):
                    v = inb[bi][d, pl.ds(u * 16, 16)]
                    idx = iota32 + (u * 16 * EMB + d)
                    plsc.store_scatter(outb[bi], [idx], v)

        @pl.when(nc > 0)
        def _():
            start_in(0, 0)

        @pl.when(nc > 1)
        def _():
            start_in(1, 1)

        def step(t, carry):
            bi = lax.rem(t, 2)
            for b in range(2):
                @pl.when(bi == b)
                def _(b=b):
                    c = wid + t * NW
                    pltpu.make_async_copy(
                        t_hbm.at[:, pl.ds(c * 128, 128)], inb[b], sin[b]
                    ).wait()

                    @pl.when(t >= 2)
                    def _(b=b):
                        co = wid + (t - 2) * NW
                        pltpu.make_async_copy(
                            outb[b],
                            out_hbm.at[pl.ds(co * (128 * EMB), 128 * EMB)],
                            sout[b],
                        ).wait()

                    shuffle(b, 128)
                    pltpu.async_copy(
                        outb[b],
                        out_hbm.at[pl.ds(c * (128 * EMB), 128 * EMB)],
                        sout[b],
                    )

                    @pl.when(t + 2 < nc)
                    def _(b=b):
                        start_in(t + 2, b)
            return carry

        lax.fori_loop(0, nc, step, 0)

        for bb in range(2):
            pltpu.make_async_copy(
                outb[bb],
                out_hbm.at[pl.ds(wid * (128 * EMB), 128 * EMB)],
                sout[bb],
            ).wait()

        @pl.when(wid == TAIL_W)
        def _():
            pltpu.sync_copy(
                tail_hbm,
                out_hbm.at[pl.ds(NFULL * 128 * EMB, TAIL * EMB)])

    return k(tableT, tail_lin)


def _sc_pool(x2d, table):
    mesh = plsc.VectorSubcoreMesh(core_axis_name="c", subcore_axis_name="s")

    @functools.partial(
        pl.kernel,
        mesh=mesh,
        compiler_params=pltpu.CompilerParams(
            use_tc_tiling_on_sc=False, needs_layout_passes=False),
        out_type=jax.ShapeDtypeStruct((BATCH, EMB), jnp.float32),
        scratch_types=[
            pltpu.VMEM((NCHUNK, CH), jnp.int32),
            pltpu.VMEM((NBUF, CH, EMB), jnp.float32),
            pltpu.VMEM((BPW, EMB), jnp.float32),
            pltpu.SemaphoreType.DMA,
            pltpu.SemaphoreType.DMA,
            pltpu.SemaphoreType.DMA,
            pltpu.SemaphoreType.DMA,
        ],
    )
    def k(x_hbm, table_hbm, out_hbm, idx_v, rows_v, acc_v, s0, s1, s2, s3):
        sems = (s0, s1, s2, s3)
        wid = lax.axis_index("s") * NC + lax.axis_index("c")
        pltpu.sync_copy(x_hbm.at[pl.ds(wid * NCHUNK, NCHUNK)], idx_v)

        def start(ci, b):
            pltpu.async_copy(table_hbm.at[idx_v.at[ci]], rows_v.at[b], sems[b])

        for b in range(NBUF):
            start(b, b)

        def group(g, carry):
            for sl in range(2):
                i = g * 2 + sl
                a0 = jnp.zeros((16,), jnp.float32)
                a1 = jnp.zeros((16,), jnp.float32)
                for j in range(CPS):
                    b = sl * CPS + j
                    ci = g * NBUF + b
                    pltpu.make_async_copy(
                        table_hbm.at[idx_v.at[ci]], rows_v.at[b], sems[b]
                    ).wait()

                    def rbody(rr, c, _b=b):
                        c0, c1, c2, c3 = c
                        r = rr * 5
                        c0 = c0 + rows_v[_b, r, pl.ds(0, 16)]
                        c1 = c1 + rows_v[_b, r, pl.ds(16, 16)]
                        c2 = c2 + rows_v[_b, r + 1, pl.ds(0, 16)]
                        c3 = c3 + rows_v[_b, r + 1, pl.ds(16, 16)]
                        c0 = c0 + rows_v[_b, r + 2, pl.ds(0, 16)]
                        c1 = c1 + rows_v[_b, r + 2, pl.ds(16, 16)]
                        c2 = c2 + rows_v[_b, r + 3, pl.ds(0, 16)]
                        c3 = c3 + rows_v[_b, r + 3, pl.ds(16, 16)]
                        c0 = c0 + rows_v[_b, r + 4, pl.ds(0, 16)]
                        c1 = c1 + rows_v[_b, r + 4, pl.ds(16, 16)]
                        return (c0, c1, c2, c3)

                    z = jnp.zeros((16,), jnp.float32)
                    a0, a1, a2, a3 = lax.fori_loop(
                        0, CH // 5, rbody, (a0, a1, z, z))
                    a0 = a0 + a2
                    a1 = a1 + a3

                    nci = ci + NBUF

                    @pl.when(nci < NCHUNK)
                    def _(nci=nci, b=b):
                        start(nci, b)

                acc_v[i, pl.ds(0, 16)] = a0
                acc_v[i, pl.ds(16, 16)] = a1
            return carry

        lax.fori_loop(0, NGROUP, group, 0)
        pltpu.sync_copy(acc_v, out_hbm.at[pl.ds(wid * BPW, BPW)])

    return k(x2d, table)


def _tc_classify(pooled_sum, W, b2d):
    def body(p_ref, w_ref, b_ref, o_ref):
        p = p_ref[...]
        w = w_ref[...]
        acc = lax.dot_general(
            p, w, (((1,), (1,)), ((), ())),
            preferred_element_type=jnp.float32,
        )
        o_ref[...] = acc * (1.0 / MAXLEN) + b_ref[...]

    return pl.pallas_call(
        body,
        out_shape=jax.ShapeDtypeStruct((BATCH, LABELS), jnp.float32),
    )(pooled_sum, W, b2d)


def kernel(x, table, W, b):
    x2d = x.reshape(BATCH * CPS, CH).astype(jnp.int32)
    tail_lin = table[VOCAB - TAIL:, :].reshape(TAIL * EMB)
    table_lin = _sc_detile(table, tail_lin).reshape(VOCAB, EMB)
    pooled_sum = _sc_pool(x2d, table_lin)
    return _tc_classify(pooled_sum, W, b.reshape(1, LABELS))


# speedup vs baseline: 2.8676x; 1.6773x over previous
import functools

import jax
import jax.numpy as jnp
from jax import lax
from jax.experimental import pallas as pl
from jax.experimental.pallas import tpu as pltpu
from jax.experimental.pallas import tpu_sc as plsc

BATCH = 4096
MAXLEN = 200
EMB = 32
LABELS = 4
VOCAB = 1000000

NC = 2
NS = 16
NW = NC * NS
BPW = BATCH // NW
CH = 100
CPS = MAXLEN // CH
NCHUNK = BPW * CPS
NBUF = 4
NGROUP = NCHUNK // NBUF


def _tc_detile_pad(table):
    tableT = table.T
    C = 4096
    nblk = (VOCAB + C - 1) // C
    eye = jnp.eye(EMB, dtype=jnp.float32)

    def body(t_ref, eye_ref, o_ref):
        tt = lax.dot_general(
            t_ref[...], eye_ref[...], (((0,), (0,)), ((), ())),
            preferred_element_type=jnp.float32,
        )
        o_ref[:, 0:EMB] = tt

    return pl.pallas_call(
        body,
        grid=(nblk,),
        in_specs=[
            pl.BlockSpec((EMB, C), lambda i: (0, i)),
            pl.BlockSpec((EMB, EMB), lambda i: (0, 0)),
        ],
        out_specs=pl.BlockSpec((C, 128), lambda i: (i, 0)),
        out_shape=jax.ShapeDtypeStruct((VOCAB, 128), jnp.float32),
    )(tableT, eye)


def _sc_pool(x2d, table):
    mesh = plsc.VectorSubcoreMesh(core_axis_name="c", subcore_axis_name="s")

    @functools.partial(
        pl.kernel,
        mesh=mesh,
        compiler_params=pltpu.CompilerParams(use_tc_tiling_on_sc=False),
        out_type=jax.ShapeDtypeStruct((BATCH, EMB), jnp.float32),
        scratch_types=[
            pltpu.VMEM((NCHUNK, CH), jnp.int32),
            pltpu.VMEM((NBUF, CH, EMB), jnp.float32),
            pltpu.VMEM((BPW, EMB), jnp.float32),
            pltpu.SemaphoreType.DMA,
            pltpu.SemaphoreType.DMA,
            pltpu.SemaphoreType.DMA,
            pltpu.SemaphoreType.DMA,
        ],
    )
    def k(x_hbm, table_hbm, out_hbm, idx_v, rows_v, acc_v, s0, s1, s2, s3):
        sems = (s0, s1, s2, s3)
        wid = lax.axis_index("s") * NC + lax.axis_index("c")
        pltpu.sync_copy(x_hbm.at[pl.ds(wid * NCHUNK, NCHUNK)], idx_v)

        def start(ci, b):
            pltpu.async_copy(table_hbm.at[idx_v.at[ci]], rows_v.at[b], sems[b])

        for b in range(NBUF):
            start(b, b)

        def group(g, carry):
            for sl in range(2):
                i = g * 2 + sl
                a0 = jnp.zeros((16,), jnp.float32)
                a1 = jnp.zeros((16,), jnp.float32)
                for j in range(CPS):
                    b = sl * CPS + j
                    ci = g * NBUF + b
                    pltpu.make_async_copy(
                        table_hbm.at[idx_v.at[ci]], rows_v.at[b], sems[b]
                    ).wait()

                    def rbody(rr, c, _b=b):
                        c0, c1, c2, c3 = c
                        r = rr * 5
                        c0 = c0 + rows_v[_b, r, pl.ds(0, 16)]
                        c1 = c1 + rows_v[_b, r, pl.ds(16, 16)]
                        c2 = c2 + rows_v[_b, r + 1, pl.ds(0, 16)]
                        c3 = c3 + rows_v[_b, r + 1, pl.ds(16, 16)]
                        c0 = c0 + rows_v[_b, r + 2, pl.ds(0, 16)]
                        c1 = c1 + rows_v[_b, r + 2, pl.ds(16, 16)]
                        c2 = c2 + rows_v[_b, r + 3, pl.ds(0, 16)]
                        c3 = c3 + rows_v[_b, r + 3, pl.ds(16, 16)]
                        c0 = c0 + rows_v[_b, r + 4, pl.ds(0, 16)]
                        c1 = c1 + rows_v[_b, r + 4, pl.ds(16, 16)]
                        return (c0, c1, c2, c3)

                    z = jnp.zeros((16,), jnp.float32)
                    a0, a1, a2, a3 = lax.fori_loop(
                        0, CH // 5, rbody, (a0, a1, z, z))
                    a0 = a0 + a2
                    a1 = a1 + a3

                    nci = ci + NBUF

                    @pl.when(nci < NCHUNK)
                    def _(nci=nci, b=b):
                        start(nci, b)

                acc_v[i, pl.ds(0, 16)] = a0
                acc_v[i, pl.ds(16, 16)] = a1
            return carry

        lax.fori_loop(0, NGROUP, group, 0)
        pltpu.sync_copy(acc_v, out_hbm.at[pl.ds(wid * BPW, BPW)])

    return k(x2d, table)


def _tc_classify(pooled_sum, W, b2d):
    def body(p_ref, w_ref, b_ref, o_ref):
        acc = lax.dot_general(
            p_ref[...], w_ref[...], (((1,), (1,)), ((), ())),
            preferred_element_type=jnp.float32,
        )
        o_ref[...] = acc * (1.0 / MAXLEN) + b_ref[...]

    return pl.pallas_call(
        body,
        out_shape=jax.ShapeDtypeStruct((BATCH, LABELS), jnp.float32),
    )(pooled_sum, W, b2d)


def kernel(x, table, W, b):
    x2d = (x.reshape(BATCH * CPS, CH) * 4).astype(jnp.int32)
    table_lin = _tc_detile_pad(table).reshape(4 * VOCAB, EMB)
    pooled_sum = _sc_pool(x2d, table_lin)
    return _tc_classify(pooled_sum, W, b.reshape(1, LABELS))


# speedup vs baseline: 3.4798x; 1.2135x over previous
import functools

import jax
import jax.numpy as jnp
from jax import lax
from jax.experimental import pallas as pl
from jax.experimental.pallas import tpu as pltpu
from jax.experimental.pallas import tpu_sc as plsc

BATCH = 4096
MAXLEN = 200
EMB = 32
LABELS = 4
VOCAB = 1000000

NC = 2
NS = 16
NW = NC * NS
BPW = BATCH // NW
CH = 100
CPS = MAXLEN // CH
NCHUNK = BPW * CPS
NBUF = 4
NGROUP = NCHUNK // NBUF


def _tc_detile_pad(table):
    tableT = table.T
    C = 8192
    nblk = (VOCAB + C - 1) // C
    eye = jnp.eye(EMB, dtype=jnp.float32)

    def body(t_ref, eye_ref, o_ref):
        tt = lax.dot_general(
            t_ref[...], eye_ref[...], (((0,), (0,)), ((), ())),
            preferred_element_type=jnp.float32,
        )
        o_ref[:, 0:EMB] = tt

    return pl.pallas_call(
        body,
        grid=(nblk,),
        in_specs=[
            pl.BlockSpec((EMB, C), lambda i: (0, i)),
            pl.BlockSpec((EMB, EMB), lambda i: (0, 0)),
        ],
        out_specs=pl.BlockSpec((C, 128), lambda i: (i, 0)),
        out_shape=jax.ShapeDtypeStruct((VOCAB, 128), jnp.float32),
    )(tableT, eye)


def _sc_pool(x2d, table):
    mesh = plsc.VectorSubcoreMesh(core_axis_name="c", subcore_axis_name="s")

    @functools.partial(
        pl.kernel,
        mesh=mesh,
        compiler_params=pltpu.CompilerParams(use_tc_tiling_on_sc=False),
        out_type=jax.ShapeDtypeStruct((BATCH, EMB), jnp.float32),
        scratch_types=[
            pltpu.VMEM((NCHUNK, CH), jnp.int32),
            pltpu.VMEM((NBUF, CH, EMB), jnp.float32),
            pltpu.VMEM((BPW, EMB), jnp.float32),
            pltpu.SemaphoreType.DMA,
            pltpu.SemaphoreType.DMA,
            pltpu.SemaphoreType.DMA,
            pltpu.SemaphoreType.DMA,
        ],
    )
    def k(x_hbm, table_hbm, out_hbm, idx_v, rows_v, acc_v, s0, s1, s2, s3):
        sems = (s0, s1, s2, s3)
        wid = lax.axis_index("s") * NC + lax.axis_index("c")
        pltpu.sync_copy(x_hbm.at[pl.ds(wid * NCHUNK, NCHUNK)], idx_v)

        def start(ci, b):
            pltpu.async_copy(table_hbm.at[idx_v.at[ci]], rows_v.at[b], sems[b])

        for b in range(NBUF):
            start(b, b)

        def group(g, carry):
            for sl in range(2):
                i = g * 2 + sl
                a0 = jnp.zeros((16,), jnp.float32)
                a1 = jnp.zeros((16,), jnp.float32)
                for j in range(CPS):
                    b = sl * CPS + j
                    ci = g * NBUF + b
                    pltpu.make_async_copy(
                        table_hbm.at[idx_v.at[ci]], rows_v.at[b], sems[b]
                    ).wait()

                    def rbody(rr, c, _b=b):
                        c0, c1, c2, c3 = c
                        r = rr * 5
                        c0 = c0 + rows_v[_b, r, pl.ds(0, 16)]
                        c1 = c1 + rows_v[_b, r, pl.ds(16, 16)]
                        c2 = c2 + rows_v[_b, r + 1, pl.ds(0, 16)]
                        c3 = c3 + rows_v[_b, r + 1, pl.ds(16, 16)]
                        c0 = c0 + rows_v[_b, r + 2, pl.ds(0, 16)]
                        c1 = c1 + rows_v[_b, r + 2, pl.ds(16, 16)]
                        c2 = c2 + rows_v[_b, r + 3, pl.ds(0, 16)]
                        c3 = c3 + rows_v[_b, r + 3, pl.ds(16, 16)]
                        c0 = c0 + rows_v[_b, r + 4, pl.ds(0, 16)]
                        c1 = c1 + rows_v[_b, r + 4, pl.ds(16, 16)]
                        return (c0, c1, c2, c3)

                    z = jnp.zeros((16,), jnp.float32)
                    a0, a1, a2, a3 = lax.fori_loop(
                        0, CH // 5, rbody, (a0, a1, z, z))
                    a0 = a0 + a2
                    a1 = a1 + a3

                    nci = ci + NBUF

                    @pl.when(nci < NCHUNK)
                    def _(nci=nci, b=b):
                        start(nci, b)

                acc_v[i, pl.ds(0, 16)] = a0
                acc_v[i, pl.ds(16, 16)] = a1
            return carry

        lax.fori_loop(0, NGROUP, group, 0)
        pltpu.sync_copy(acc_v, out_hbm.at[pl.ds(wid * BPW, BPW)])

    return k(x2d, table)


def _tc_classify(pooled_sum, W, b2d):
    def body(p_ref, w_ref, b_ref, o_ref):
        acc = lax.dot_general(
            p_ref[...], w_ref[...], (((1,), (1,)), ((), ())),
            preferred_element_type=jnp.float32,
        )
        o_ref[...] = acc * (1.0 / MAXLEN) + b_ref[...]

    return pl.pallas_call(
        body,
        out_shape=jax.ShapeDtypeStruct((BATCH, LABELS), jnp.float32),
    )(pooled_sum, W, b2d)


def kernel(x, table, W, b):
    x2d = (x.reshape(BATCH * CPS, CH) * 4).astype(jnp.int32)
    table_lin = _tc_detile_pad(table).reshape(4 * VOCAB, EMB)
    pooled_sum = _sc_pool(x2d, table_lin)
    return _tc_classify(pooled_sum, W, b.reshape(1, LABELS))
